# Initial kernel scaffold; baseline (speedup 1.0000x reference)
import jax
import jax.numpy as jnp
from jax.experimental import pallas as pl


def kernel(xytp, features, pe_w1, pe_b1, bn_g, bn_b, pe_w2, pe_b2, gt_w, gt_b, ln_g, ln_b):
    raise NotImplementedError("write your pallas kernel here")


# speedup vs baseline: 9.9813x; 9.9813x over previous
import functools
import math

import jax
import jax.numpy as jnp
from jax import lax
from jax.experimental import pallas as pl
from jax.experimental.pallas import tpu as pltpu
from jax.experimental.pallas import tpu_sc as plsc

F32 = jnp.float32
I32 = jnp.int32

B = 4
N = 8192
M = 1024
KG = 16
NCH = 16
C = N // NCH
NW = 32


def _fps_body(pts_ref, out_ref):
    pts = [pts_ref[c * B:(c + 1) * B, :] for c in range(4)]
    lane = lax.broadcasted_iota(I32, (B, N), 1)
    col = lax.broadcasted_iota(I32, (4 * B, M), 1)

    def body(i, carry):
        dists, nxt, sampled = carry
        mask = lane == nxt
        sels = [jnp.sum(jnp.where(mask, p, 0.0), axis=1, keepdims=True)
                for p in pts]
        sel_col = jnp.concatenate(sels, axis=0)
        sampled = jnp.where(col == i, sel_col, sampled)
        d = (pts[0] - sels[0]) ** 2 + (pts[1] - sels[1]) ** 2
        d = d + (pts[2] - sels[2]) ** 2
        dists = jnp.minimum(dists, d)
        mx = jnp.max(dists, axis=1, keepdims=True)
        cand = jnp.where(dists == mx, lane, N)
        nxt = jnp.min(cand, axis=1, keepdims=True)
        return dists, nxt, sampled

    init = (jnp.full((B, N), jnp.inf, F32), jnp.zeros((B, 1), I32),
            jnp.zeros((4 * B, M), F32))
    _, _, sampled = lax.fori_loop(0, M, body, init)
    out_ref[...] = sampled


def _fps(xytp_cm):
    return pl.pallas_call(
        _fps_body,
        out_shape=jax.ShapeDtypeStruct((4 * B, M), F32),
    )(xytp_cm)


def _gt_body(f_ref, w_ref, b_ref, o_ref):
    y = jnp.dot(f_ref[...], w_ref[...].T,
                preferred_element_type=F32,
                precision=lax.Precision.HIGHEST) + b_ref[...]
    o_ref[...] = jnp.concatenate(
        [y, jnp.zeros((y.shape[0], 32), F32)], axis=1)


def _gt_mm(features2d, gt_w, gt_b):
    rows = B * N
    blk = 2048
    return pl.pallas_call(
        _gt_body,
        grid=(rows // blk,),
        in_specs=[
            pl.BlockSpec((blk, 32), lambda i: (i, 0)),
            pl.BlockSpec((96, 32), lambda i: (0, 0)),
            pl.BlockSpec((1, 96), lambda i: (0, 0)),
        ],
        out_specs=pl.BlockSpec((blk, 128), lambda i: (i, 0)),
        out_shape=jax.ShapeDtypeStruct((rows, 128), F32),
    )(features2d, gt_w, gt_b.reshape(1, 96))


def _knn_body(x_ref, s_ref, inv_ref, pair_ref, cv_ref, ci_ref):
    b = pl.program_id(0)
    ch = pl.program_id(1)
    pts = x_ref[0]
    samp = s_ref[0]
    d = None
    for c in range(3):
        pc = pts[:, c:c + 1]
        sc = samp[c:c + 1, :]
        q = (pc - sc) * (pc - sc)
        d = q if d is None else d + q

    lane = lax.broadcasted_iota(I32, (C, M), 1)
    row = lax.broadcasted_iota(I32, (C, M), 0)

    dw = d
    for j in range(KG):
        mn = jnp.min(dw, axis=1, keepdims=True)
        cand = jnp.where(dw == mn, lane, M)
        idx = jnp.min(cand, axis=1, keepdims=True)
        inv_ref[0, :, j:j + 1] = idx + b * M
        dw = jnp.where(lane == idx, jnp.inf, dw)

    dw = d
    for j in range(KG):
        mn = jnp.min(dw, axis=0, keepdims=True)
        cand = jnp.where(dw == mn, row, C)
        ridx = jnp.min(cand, axis=0, keepdims=True)
        cv_ref[pl.ds(ch * KG + j, 1), :] = mn
        ci_ref[pl.ds(ch * KG + j, 1), :] = ridx + ch * C
        dw = jnp.where(row == ridx, jnp.inf, dw)

    @pl.when(ch == NCH - 1)
    def _merge():
        vals = cv_ref[...]
        idxs = ci_ref[...]
        row2 = lax.broadcasted_iota(I32, (NCH * KG, M), 0)
        v = vals
        for j in range(KG):
            mn = jnp.min(v, axis=0, keepdims=True)
            cand = jnp.where(v == mn, row2, NCH * KG)
            r = jnp.min(cand, axis=0, keepdims=True)
            sel = jnp.sum(jnp.where(row2 == r, idxs, 0), axis=0, keepdims=True)
            pair_ref[0, j:j + 1, :] = sel + b * N
            v = jnp.where(row2 == r, jnp.inf, v)


def _knn(xytp, samp_bm):
    return pl.pallas_call(
        _knn_body,
        grid=(B, NCH),
        in_specs=[
            pl.BlockSpec((1, C, 4), lambda b, ch: (b, ch, 0)),
            pl.BlockSpec((1, 4, M), lambda b, ch: (b, 0, 0)),
        ],
        out_specs=[
            pl.BlockSpec((1, C, KG), lambda b, ch: (b, ch, 0)),
            pl.BlockSpec((1, KG, M), lambda b, ch: (b, 0, 0)),
        ],
        out_shape=[
            jax.ShapeDtypeStruct((B, N, KG), I32),
            jax.ShapeDtypeStruct((B, KG, M), I32),
        ],
        scratch_shapes=[
            pltpu.VMEM((NCH * KG, M), F32),
            pltpu.VMEM((NCH * KG, M), I32),
        ],
    )(xytp, samp_bm.reshape(B, 4, M))


def _pool_body(gt_hbm, pidx_hbm, sx_hbm, out_hbm, idx_v, rows_v, sxv, out_v,
               sem):
    wid = lax.axis_index("s") * 2 + lax.axis_index("c")
    rows_per_w = (B * M * KG) // NW
    m_per_w = rows_per_w // KG
    groups_per_chunk = 32
    chunk_rows = groups_per_chunk * KG
    base = wid * rows_per_w
    pltpu.sync_copy(pidx_hbm.at[pl.ds(base, rows_per_w)], idx_v)
    pltpu.sync_copy(sx_hbm.at[pl.ds(wid * m_per_w, m_per_w)], sxv)

    def chunk(ci, _):
        pltpu.async_copy(
            gt_hbm.at[idx_v.at[pl.ds(ci * chunk_rows, chunk_rows)]],
            rows_v, sem).wait()

        def group(g, _):
            accs = [rows_v[g * KG, pl.ds(32 + 16 * j, 16)] for j in range(4)]
            for k in range(1, KG):
                for j in range(4):
                    accs[j] = jnp.maximum(
                        accs[j], rows_v[g * KG + k, pl.ds(32 + 16 * j, 16)])
            for j in range(4):
                out_v[ci * groups_per_chunk + g, pl.ds(16 * j, 16)] = accs[j]
            return 0

        lax.fori_loop(0, groups_per_chunk, group, 0)
        return 0

    lax.fori_loop(0, rows_per_w // chunk_rows, chunk, 0)

    def sxcopy(r, _):
        out_v[r, pl.ds(64, 16)] = sxv[r, pl.ds(0, 16)]
        out_v[r, pl.ds(80, 16)] = jnp.zeros((16,), F32)
        out_v[r, pl.ds(96, 16)] = jnp.zeros((16,), F32)
        out_v[r, pl.ds(112, 16)] = jnp.zeros((16,), F32)
        return 0

    lax.fori_loop(0, m_per_w, sxcopy, 0)
    pltpu.sync_copy(out_v, out_hbm.at[pl.ds(wid * m_per_w, m_per_w)])


def _pool(gt, pidx, sx_tab):
    mesh = plsc.VectorSubcoreMesh(core_axis_name="c", subcore_axis_name="s")
    rows_per_w = (B * M * KG) // NW
    fn = pl.kernel(
        _pool_body,
        out_type=jax.ShapeDtypeStruct((B * M, 128), F32),
        mesh=mesh,
        scratch_types=[
            pltpu.VMEM((rows_per_w,), I32),
            pltpu.VMEM((512, 128), F32),
            pltpu.VMEM((rows_per_w // KG, 16), F32),
            pltpu.VMEM((rows_per_w // KG, 128), F32),
            pltpu.SemaphoreType.DMA,
        ],
    )
    return fn(gt, pidx, sx_tab)


def _expand_body(pool_hbm, iidx_hbm, pg_hbm, idx_v, pgv, sem):
    wid = lax.axis_index("s") * 2 + lax.axis_index("c")
    rows_per_w = (B * N * KG) // NW
    chunk_rows = 512
    base = wid * rows_per_w
    pltpu.sync_copy(iidx_hbm.at[pl.ds(base, rows_per_w)], idx_v)

    def chunk(ci, _):
        sl = idx_v.at[pl.ds(ci * chunk_rows, chunk_rows)]
        pltpu.async_copy(pool_hbm.at[sl], pgv, sem).wait()
        pltpu.sync_copy(pgv,
                        pg_hbm.at[pl.ds(base + ci * chunk_rows, chunk_rows)])
        return 0

    lax.fori_loop(0, rows_per_w // chunk_rows, chunk, 0)


def _expand(pooled, iidx):
    mesh = plsc.VectorSubcoreMesh(core_axis_name="c", subcore_axis_name="s")
    rows_per_w = (B * N * KG) // NW
    fn = pl.kernel(
        _expand_body,
        out_type=jax.ShapeDtypeStruct((B * N * KG, 128), F32),
        mesh=mesh,
        scratch_types=[
            pltpu.VMEM((rows_per_w,), I32),
            pltpu.VMEM((512, 128), F32),
            pltpu.SemaphoreType.DMA,
        ],
    )
    return fn(pooled, iidx)


def _stats_body(x_ref, sg_ref, w1_ref, b1_ref, o_ref):
    i = pl.program_id(0)
    x = x_ref[0]
    s = sg_ref[0][:, 64:68]
    xb = jnp.broadcast_to(x.reshape(C, 1, 4), (C, KG, 4)).reshape(C * KG, 4)
    rel = xb - s
    y = jnp.dot(rel, w1_ref[...].T, preferred_element_type=F32,
                precision=lax.Precision.HIGHEST) + b1_ref[...]

    @pl.when(i == 0)
    def _init():
        o_ref[...] = jnp.zeros_like(o_ref)

    o_ref[0:1, 0:4] += jnp.sum(y, axis=0, keepdims=True)
    o_ref[1:2, 0:4] += jnp.sum(y * y, axis=0, keepdims=True)


def _stats(xytp, pg, pe_w1, pe_b1):
    steps = (B * N) // C
    return pl.pallas_call(
        _stats_body,
        grid=(steps,),
        in_specs=[
            pl.BlockSpec((1, C, 4), lambda i: (i, 0, 0)),
            pl.BlockSpec((1, C * KG, 128), lambda i: (i, 0, 0)),
            pl.BlockSpec((4, 4), lambda i: (0, 0)),
            pl.BlockSpec((1, 4), lambda i: (0, 0)),
        ],
        out_specs=pl.BlockSpec((8, 128), lambda i: (0, 0)),
        out_shape=jax.ShapeDtypeStruct((8, 128), F32),
    )(xytp.reshape(steps, C, 4), pg.reshape(steps, C * KG, 128),
      pe_w1, pe_b1.reshape(1, 4))


def _attend_body(x_ref, g_ref, pg_ref, mv_ref, bng_ref, bnb_ref,
                 w1_ref, b1_ref, w2_ref, b2_ref, lng_ref, lnb_ref, o_ref):
    x = x_ref[0]
    g = g_ref[0]
    pgc = pg_ref[0]

    xb = jnp.broadcast_to(x.reshape(C, 1, 4), (C, KG, 4)).reshape(C * KG, 4)
    rel = xb - pgc[:, 64:68]
    y1 = jnp.dot(rel, w1_ref[...].T, preferred_element_type=F32,
                 precision=lax.Precision.HIGHEST) + b1_ref[...]
    mean = mv_ref[0:1, :]
    var = mv_ref[1:2, :]
    yn = (y1 - mean) / jnp.sqrt(var + 1e-5) * bng_ref[...] + bnb_ref[...]
    yr = jnp.maximum(yn, 0.0)
    delta = jnp.dot(yr, w2_ref[...].T, preferred_element_type=F32,
                    precision=lax.Precision.HIGHEST) + b2_ref[...]

    varphi = g[:, 0:32]
    vb = jnp.broadcast_to(varphi.reshape(C, 1, 32),
                          (C, KG, 32)).reshape(C * KG, 32)
    xpre = vb - pgc[:, 0:32] + delta
    mu = jnp.mean(xpre, axis=1, keepdims=True)
    vv = jnp.mean((xpre - mu) ** 2, axis=1, keepdims=True)
    ln = (xpre - mu) / jnp.sqrt(vv + 1e-5) * lng_ref[...] + lnb_ref[...]
    pre = (ln / math.sqrt(32.0)).reshape(C, KG, 32)

    mx = jnp.max(pre, axis=1, keepdims=True)
    e = jnp.exp(pre - mx)
    ssum = jnp.sum(e, axis=1, keepdims=True)
    attn = e / ssum
    val = (pgc[:, 32:64] + delta).reshape(C, KG, 32)
    o_ref[0] = jnp.sum(attn * val, axis=1)


def _attend(xytp, gt, pg, mv, bn_g, bn_b, pe_w1, pe_b1, pe_w2, pe_b2,
            ln_g, ln_b):
    steps = (B * N) // C
    return pl.pallas_call(
        _attend_body,
        grid=(steps,),
        in_specs=[
            pl.BlockSpec((1, C, 4), lambda i: (i, 0, 0)),
            pl.BlockSpec((1, C, 128), lambda i: (i, 0, 0)),
            pl.BlockSpec((1, C * KG, 128), lambda i: (i, 0, 0)),
            pl.BlockSpec((2, 4), lambda i: (0, 0)),
            pl.BlockSpec((1, 4), lambda i: (0, 0)),
            pl.BlockSpec((1, 4), lambda i: (0, 0)),
            pl.BlockSpec((4, 4), lambda i: (0, 0)),
            pl.BlockSpec((1, 4), lambda i: (0, 0)),
            pl.BlockSpec((32, 4), lambda i: (0, 0)),
            pl.BlockSpec((1, 32), lambda i: (0, 0)),
            pl.BlockSpec((1, 32), lambda i: (0, 0)),
            pl.BlockSpec((1, 32), lambda i: (0, 0)),
        ],
        out_specs=pl.BlockSpec((1, C, 32), lambda i: (i, 0, 0)),
        out_shape=jax.ShapeDtypeStruct((steps, C, 32), F32),
    )(xytp.reshape(steps, C, 4), gt.reshape(steps, C, 128),
      pg.reshape(steps, C * KG, 128),
      mv, bn_g.reshape(1, 4), bn_b.reshape(1, 4), pe_w1,
      pe_b1.reshape(1, 4), pe_w2, pe_b2.reshape(1, 32),
      ln_g.reshape(1, 32), ln_b.reshape(1, 32))


def kernel(xytp, features, pe_w1, pe_b1, bn_g, bn_b, pe_w2, pe_b2,
           gt_w, gt_b, ln_g, ln_b):
    xytp_cm = jnp.transpose(xytp, (2, 0, 1)).reshape(4 * B, N)
    samp_cm = _fps(xytp_cm)
    s_b = samp_cm.reshape(4, B, M)
    samp_bm = jnp.transpose(s_b, (1, 0, 2)).reshape(B * 4, M)
    sx_tab = jnp.pad(jnp.transpose(s_b, (1, 2, 0)),
                     ((0, 0), (0, 0), (0, 12))).reshape(B * M, 16)

    gt = _gt_mm(features.reshape(B * N, 32), gt_w, gt_b)
    inv_g, pair_t = _knn(xytp, samp_bm)
    pidx = jnp.transpose(pair_t, (0, 2, 1)).reshape(B * M * KG)
    iidx = inv_g.reshape(B * N * KG)

    pooled = _pool(gt, pidx, sx_tab)
    pg = _expand(pooled, iidx)

    sums = _stats(xytp, pg, pe_w1, pe_b1)
    cnt = float(B * N * KG)
    mean = sums[0:1, 0:4] / cnt
    var = sums[1:2, 0:4] / cnt - (sums[0:1, 0:4] / cnt) ** 2
    mv = jnp.concatenate([mean, var], axis=0)

    out = _attend(xytp, gt, pg, mv, bn_g, bn_b, pe_w1, pe_b1,
                  pe_w2, pe_b2, ln_g, ln_b)
    return out.reshape(B, N, 32)
